# baseline (speedup 1.0000x reference)
import jax
import jax.numpy as jnp
from jax import lax
from jax.experimental import pallas as pl
from jax.experimental.pallas import tpu as pltpu

_D = 4096
_SEQ = 8192
_BATCH = 4
_NE = 32
_H1 = 2048
_SBLK = 512
_NS = _SEQ // _SBLK
_INV_SEQ = 1.0 / _SEQ


def _fused_kernel(x_ref, w1_ref, b1_ref, g1_ref, be1_ref,
                  w2_ref, b2_ref, g2_ref, be2_ref,
                  w3_ref, b3_ref, o_ref, acc_ref, h1_ref):
    b = pl.program_id(0)
    s = pl.program_id(1)
    part = jnp.sum(x_ref[...], axis=1)

    @pl.when(s == 0)
    def _():
        acc_ref[pl.ds(b, 1), :] = part

    @pl.when(s > 0)
    def _():
        acc_ref[pl.ds(b, 1), :] += part

    @pl.when(s == _NS - 1)
    def _():
        pooled_b = acc_ref[pl.ds(b, 1), :] * _INV_SEQ
        h1_ref[pl.ds(b, 1), :] = lax.dot_general(
            pooled_b, w1_ref[...], (((1,), (1,)), ((), ())),
            preferred_element_type=jnp.float32)

    @pl.when((b == _BATCH - 1) & (s == _NS - 1))
    def _():
        def _ln(h, g, bb, eps=1e-5):
            m = jnp.mean(h, axis=-1, keepdims=True)
            v = jnp.mean((h - m) ** 2, axis=-1, keepdims=True)
            return (h - m) / jnp.sqrt(v + eps) * g + bb

        h = h1_ref[...] + b1_ref[...]
        h = jax.nn.relu(_ln(h, g1_ref[...], be1_ref[...]))
        h = lax.dot_general(h, w2_ref[...], (((1,), (1,)), ((), ())),
                            preferred_element_type=jnp.float32) + b2_ref[...]
        h = jax.nn.relu(_ln(h, g2_ref[...], be2_ref[...]))
        scores = lax.dot_general(h, w3_ref[...], (((1,), (1,)), ((), ())),
                                 preferred_element_type=jnp.float32) + b3_ref[...]

        scaled = scores - jnp.max(scores, axis=-1, keepdims=True)
        e = jnp.exp(scaled - jnp.max(scaled, axis=-1, keepdims=True))
        probs = e / jnp.sum(e, axis=-1, keepdims=True)

        pa = probs[:, :, None]
        pb = probs[:, None, :]
        ii = lax.broadcasted_iota(jnp.int32, (1, _NE, _NE), 1)
        jj = lax.broadcasted_iota(jnp.int32, (1, _NE, _NE), 2)
        beats = (pb > pa) | ((pb == pa) & (jj < ii))
        nbeat = jnp.sum(beats.astype(jnp.int32), axis=-1)
        o_ref[...] = (nbeat < 8).astype(jnp.float32)


def kernel(x, W1, b1, g1, be1, W2, b2, g2, be2, W3, b3):
    const = lambda shape: pl.BlockSpec(shape, lambda b, s: tuple(0 for _ in shape))
    return pl.pallas_call(
        _fused_kernel,
        grid=(_BATCH, _NS),
        in_specs=[pl.BlockSpec((1, _SBLK, _D), lambda b, s: (b, s, 0)),
                  const(W1.shape), const(b1.shape), const(g1.shape),
                  const(be1.shape), const(W2.shape), const(b2.shape),
                  const(g2.shape), const(be2.shape), const(W3.shape),
                  const(b3.shape)],
        out_specs=const((_BATCH, _NE)),
        out_shape=jax.ShapeDtypeStruct((_BATCH, _NE), jnp.float32),
        scratch_shapes=[pltpu.VMEM((_BATCH, _D), jnp.float32),
                        pltpu.VMEM((_BATCH, _H1), jnp.float32)],
    )(x, W1, b1, g1, be1, W2, b2, g2, be2, W3, b3)


# speedup vs baseline: 1.0603x; 1.0603x over previous
import jax
import jax.numpy as jnp
from jax import lax
from jax.experimental import pallas as pl
from jax.experimental.pallas import tpu as pltpu

_D = 4096
_SEQ = 8192
_BATCH = 4
_NE = 32
_SBLK = 256
_NS = _SEQ // _SBLK // 2


def _fused_kernel(xa_ref, xb_ref, w1_ref, b1_ref, g1_ref, be1_ref,
                  w2_ref, b2_ref, g2_ref, be2_ref,
                  w3_ref, b3_ref, o_ref, acc_ref):
    b = pl.program_id(0)
    s = pl.program_id(1)
    part = jnp.sum(xa_ref[...], axis=1) + jnp.sum(xb_ref[...], axis=1)

    @pl.when(s == 0)
    def _():
        acc_ref[pl.ds(b, 1), :] = part

    @pl.when(s > 0)
    def _():
        acc_ref[pl.ds(b, 1), :] += part

    @pl.when((b == _BATCH - 1) & (s == _NS - 1))
    def _():
        pooled = acc_ref[...] * (1.0 / _SEQ)

        def _ln(h, g, bb, eps=1e-5):
            m = jnp.mean(h, axis=-1, keepdims=True)
            v = jnp.mean((h - m) ** 2, axis=-1, keepdims=True)
            return (h - m) / jnp.sqrt(v + eps) * g + bb

        h = lax.dot_general(pooled, w1_ref[...], (((1,), (1,)), ((), ())),
                            preferred_element_type=jnp.float32) + b1_ref[...]
        h = jax.nn.relu(_ln(h, g1_ref[...], be1_ref[...]))
        h = lax.dot_general(h, w2_ref[...], (((1,), (1,)), ((), ())),
                            preferred_element_type=jnp.float32) + b2_ref[...]
        h = jax.nn.relu(_ln(h, g2_ref[...], be2_ref[...]))
        scores = lax.dot_general(h, w3_ref[...], (((1,), (1,)), ((), ())),
                                 preferred_element_type=jnp.float32) + b3_ref[...]

        scaled = scores - jnp.max(scores, axis=-1, keepdims=True)
        e = jnp.exp(scaled - jnp.max(scaled, axis=-1, keepdims=True))
        probs = e / jnp.sum(e, axis=-1, keepdims=True)

        pa = probs[:, :, None]
        pb = probs[:, None, :]
        ii = lax.broadcasted_iota(jnp.int32, (1, _NE, _NE), 1)
        jj = lax.broadcasted_iota(jnp.int32, (1, _NE, _NE), 2)
        beats = (pb > pa) | ((pb == pa) & (jj < ii))
        nbeat = jnp.sum(beats.astype(jnp.int32), axis=-1)
        o_ref[...] = (nbeat < 8).astype(jnp.float32)


def kernel(x, W1, b1, g1, be1, W2, b2, g2, be2, W3, b3):
    const = lambda shape: pl.BlockSpec(shape, lambda b, s: tuple(0 for _ in shape))
    return pl.pallas_call(
        _fused_kernel,
        grid=(_BATCH, _NS),
        in_specs=[pl.BlockSpec((1, _SBLK, _D), lambda b, s: (b, s, 0)),
                  pl.BlockSpec((1, _SBLK, _D), lambda b, s: (b, s + _NS, 0)),
                  const(W1.shape), const(b1.shape), const(g1.shape),
                  const(be1.shape), const(W2.shape), const(b2.shape),
                  const(g2.shape), const(be2.shape), const(W3.shape),
                  const(b3.shape)],
        out_specs=const((_BATCH, _NE)),
        out_shape=jax.ShapeDtypeStruct((_BATCH, _NE), jnp.float32),
        scratch_shapes=[pltpu.VMEM((_BATCH, _D), jnp.float32)],
    )(x, x, W1, b1, g1, be1, W2, b2, g2, be2, W3, b3)


# speedup vs baseline: 1.0607x; 1.0004x over previous
import jax
import jax.numpy as jnp
from jax import lax
from jax.experimental import pallas as pl
from jax.experimental.pallas import tpu as pltpu

_D = 4096
_SEQ = 8192
_BATCH = 4
_NE = 32
_SBLK = 128
_NS = _SEQ // _SBLK // 4


def _fused_kernel(xa_ref, xb_ref, xc_ref, xd_ref, w1_ref, b1_ref, g1_ref, be1_ref,
                  w2_ref, b2_ref, g2_ref, be2_ref,
                  w3_ref, b3_ref, o_ref, acc_ref):
    b = pl.program_id(0)
    s = pl.program_id(1)
    part = (jnp.sum(xa_ref[...], axis=1) + jnp.sum(xb_ref[...], axis=1)
            + jnp.sum(xc_ref[...], axis=1) + jnp.sum(xd_ref[...], axis=1))

    @pl.when(s == 0)
    def _():
        acc_ref[pl.ds(b, 1), :] = part

    @pl.when(s > 0)
    def _():
        acc_ref[pl.ds(b, 1), :] += part

    @pl.when((b == _BATCH - 1) & (s == _NS - 1))
    def _():
        pooled = acc_ref[...] * (1.0 / _SEQ)

        def _ln(h, g, bb, eps=1e-5):
            m = jnp.mean(h, axis=-1, keepdims=True)
            v = jnp.mean((h - m) ** 2, axis=-1, keepdims=True)
            return (h - m) / jnp.sqrt(v + eps) * g + bb

        h = lax.dot_general(pooled, w1_ref[...], (((1,), (1,)), ((), ())),
                            preferred_element_type=jnp.float32) + b1_ref[...]
        h = jax.nn.relu(_ln(h, g1_ref[...], be1_ref[...]))
        h = lax.dot_general(h, w2_ref[...], (((1,), (1,)), ((), ())),
                            preferred_element_type=jnp.float32) + b2_ref[...]
        h = jax.nn.relu(_ln(h, g2_ref[...], be2_ref[...]))
        scores = lax.dot_general(h, w3_ref[...], (((1,), (1,)), ((), ())),
                                 preferred_element_type=jnp.float32) + b3_ref[...]

        scaled = scores - jnp.max(scores, axis=-1, keepdims=True)
        e = jnp.exp(scaled - jnp.max(scaled, axis=-1, keepdims=True))
        probs = e / jnp.sum(e, axis=-1, keepdims=True)

        pa = probs[:, :, None]
        pb = probs[:, None, :]
        ii = lax.broadcasted_iota(jnp.int32, (1, _NE, _NE), 1)
        jj = lax.broadcasted_iota(jnp.int32, (1, _NE, _NE), 2)
        beats = (pb > pa) | ((pb == pa) & (jj < ii))
        nbeat = jnp.sum(beats.astype(jnp.int32), axis=-1)
        o_ref[...] = (nbeat < 8).astype(jnp.float32)


def kernel(x, W1, b1, g1, be1, W2, b2, g2, be2, W3, b3):
    const = lambda shape: pl.BlockSpec(shape, lambda b, s: tuple(0 for _ in shape))
    return pl.pallas_call(
        _fused_kernel,
        grid=(_BATCH, _NS),
        in_specs=[pl.BlockSpec((1, _SBLK, _D), lambda b, s: (b, s, 0)),
                  pl.BlockSpec((1, _SBLK, _D), lambda b, s: (b, s + _NS, 0)),
                  pl.BlockSpec((1, _SBLK, _D), lambda b, s: (b, s + 2 * _NS, 0)),
                  pl.BlockSpec((1, _SBLK, _D), lambda b, s: (b, s + 3 * _NS, 0)),
                  const(W1.shape), const(b1.shape), const(g1.shape),
                  const(be1.shape), const(W2.shape), const(b2.shape),
                  const(g2.shape), const(be2.shape), const(W3.shape),
                  const(b3.shape)],
        out_specs=const((_BATCH, _NE)),
        out_shape=jax.ShapeDtypeStruct((_BATCH, _NE), jnp.float32),
        scratch_shapes=[pltpu.VMEM((_BATCH, _D), jnp.float32)],
    )(x, x, x, x, W1, b1, g1, be1, W2, b2, g2, be2, W3, b3)
